# baseline (speedup 1.0000x reference)
import functools

import jax
import jax.numpy as jnp
from jax import lax
from jax.experimental import pallas as pl
from jax.experimental.pallas import tpu as pltpu
from jax.experimental.pallas import tpu_sc as plsc

B, S = 1024, 50
N = B * S
D = 128
H = 256

NC, NS = 2, 16
NW = NC * NS
NSPLIT = 2
NH = N // NSPLIT
ROWS_PER_W = NH // NW
CHUNK = 80
NCHUNK = ROWS_PER_W // CHUNK


def _sc_gather3(tabA, tabT, tabK, idxA, idxT, idxK):
    mesh = plsc.VectorSubcoreMesh(core_axis_name="c", subcore_axis_name="s")

    NBUF = 3

    @functools.partial(
        pl.kernel,
        mesh=mesh,
        out_type=[jax.ShapeDtypeStruct((NH, D), jnp.float32)] * 3,
        scratch_types=(
            [pltpu.VMEM((NCHUNK, CHUNK), jnp.int32)] * 3
            + [pltpu.VMEM((CHUNK, D), jnp.float32)] * (3 * NBUF)
            + [pltpu.SemaphoreType.DMA] * (6 * NBUF)
            + [pltpu.VMEM_SHARED((1001, D), jnp.float32)] * 2
        ),
    )
    def k(tabA_h, tabT_h, tabK_h, idxA_h, idxT_h, idxK_h,
          outA_h, outT_h, outK_h, *scr):
        idxv = scr[0:3]
        bufs = [scr[3 + t * NBUF: 3 + (t + 1) * NBUF] for t in range(3)]
        o = 3 + 3 * NBUF
        gsem = [scr[o + t * NBUF: o + (t + 1) * NBUF] for t in range(3)]
        o += 3 * NBUF
        wsem = [scr[o + t * NBUF: o + (t + 1) * NBUF] for t in range(3)]
        spT, spK = scr[o + 3 * NBUF:]

        wid = lax.axis_index("s") * NC + lax.axis_index("c")
        base = wid * ROWS_PER_W

        @pl.when(lax.axis_index("s") == 0)
        def _stage():
            pltpu.sync_copy(tabT_h, spT)
            pltpu.sync_copy(tabK_h, spK)

        pltpu.sync_copy(idxA_h.at[wid], idxv[0])
        pltpu.sync_copy(idxT_h.at[wid], idxv[1])
        pltpu.sync_copy(idxK_h.at[wid], idxv[2])
        plsc.subcore_barrier()

        tabs = (tabA_h, spT, spK)
        outs = (outA_h, outT_h, outK_h)

        def g_cp(t, j, s):
            return pltpu.make_async_copy(
                tabs[t].at[idxv[t].at[j]], bufs[t][s], gsem[t][s])

        def w_cp(t, j, s):
            return pltpu.make_async_copy(
                bufs[t][s], outs[t].at[pl.ds(base + j * CHUNK, CHUNK)],
                wsem[t][s])

        for j in range(NBUF):
            for t in range(3):
                g_cp(t, j, j).start()
        for j in range(NCHUNK):
            s = j % NBUF
            for t in range(3):
                g_cp(t, j, s).wait()
                w_cp(t, j, s).start()
            if j + NBUF < NCHUNK:
                for t in range(3):
                    w_cp(t, j, s).wait()
                    g_cp(t, j + NBUF, s).start()
        for j in range(max(0, NCHUNK - NBUF), NCHUNK):
            for t in range(3):
                w_cp(t, j, j % NBUF).wait()

    return k(tabA, tabT, tabK, idxA, idxT, idxK)


R = 5120
G2 = NH // R


_DN_T = (((0,), (0,)), ((), ()))


def _tc_body(gA, gT, gK, ii, xel, xua, xia, xta, embI, W0, W1, W2, W3,
             cb, cg, cbeta, cW, ctb, ctg, ctbeta, out):
    eps = 1e-5
    acc = jnp.dot(gA[...], W1[...], preferred_element_type=jnp.float32)
    acc += jnp.dot(gT[...], W2[...], preferred_element_type=jnp.float32)
    acc += jnp.dot(gK[...], W3[...], preferred_element_type=jnp.float32)
    P0 = jnp.dot(embI[...], W0[...], preferred_element_type=jnp.float32)
    iif = ii[0].astype(jnp.float32)
    s0 = jnp.maximum(0.0, 1.0 - jnp.abs(iif))
    s1 = jnp.maximum(0.0, 1.0 - jnp.abs(iif - 1.0))
    s2 = jnp.maximum(0.0, 1.0 - jnp.abs(iif - 2.0))
    oh = jnp.concatenate(
        [s0, s1, s2, jnp.zeros((5, s0.shape[1]), jnp.float32)], axis=0)
    p = lax.dot_general(oh, P0, _DN_T, preferred_element_type=jnp.float32)
    x = acc + p + cb[...]
    mu = jnp.mean(x, axis=-1, keepdims=True)
    xc = x - mu
    var = jnp.mean(xc * xc, axis=-1, keepdims=True)
    cate = xc * lax.rsqrt(var + eps) * cg[...] + cbeta[...]

    xq = jnp.concatenate(
        [xel[0], xua[0], xia[0], xta[0],
         jnp.zeros((4, xel.shape[2]), jnp.float32)], axis=0)
    y = lax.dot_general(xq, cW[...], _DN_T,
                        preferred_element_type=jnp.float32) + ctb[...]
    muy = jnp.mean(y, axis=-1, keepdims=True)
    yc = y - muy
    vary = jnp.mean(yc * yc, axis=-1, keepdims=True)
    cont = yc * lax.rsqrt(vary + eps) * ctg[...] + ctbeta[...]

    out[...] = jnp.concatenate([cate, cont], axis=-1)


def kernel(data_assessmentItemID, data_testId, data_KnowledgeTag, data_elapsed,
           data_user_acc, data_item_acc, data_tag_acc, data_answerCode,
           data_mask, data_interaction, emb_interaction, emb_assessmentItemID,
           emb_testId, emb_KnowledgeTag, comb_W, comb_b, comb_ln_g, comb_ln_b,
           cont_W, cont_b, cont_ln_g, cont_ln_b):
    idxA = data_assessmentItemID.T.reshape(NSPLIT, NW, NCHUNK, CHUNK).astype(jnp.int32)
    idxT = data_testId.T.reshape(NSPLIT, NW, NCHUNK, CHUNK).astype(jnp.int32)
    idxK = data_KnowledgeTag.T.reshape(NSPLIT, NW, NCHUNK, CHUNK).astype(jnp.int32)

    halves = [
        _sc_gather3(emb_assessmentItemID, emb_testId, emb_KnowledgeTag,
                    idxA[h], idxT[h], idxK[h])
        for h in range(NSPLIT)
    ]

    ii = data_interaction.T.reshape(NSPLIT * G2, 1, R).astype(jnp.int32)
    xel = data_elapsed.T.reshape(NSPLIT * G2, 1, R)
    xua = data_user_acc.T.reshape(NSPLIT * G2, 1, R)
    xia = data_item_acc.T.reshape(NSPLIT * G2, 1, R)
    xta = data_tag_acc.T.reshape(NSPLIT * G2, 1, R)

    embI = jnp.zeros((8, D), jnp.float32).at[0:3].set(emb_interaction)
    W0 = comb_W[0:D]
    W1 = comb_W[D:2 * D]
    W2 = comb_W[2 * D:3 * D]
    W3 = comb_W[3 * D:4 * D]
    cb = comb_b.reshape(1, H)
    cg = comb_ln_g.reshape(1, H)
    cbeta = comb_ln_b.reshape(1, H)
    cWp = jnp.zeros((8, H), jnp.float32).at[0:4].set(cont_W)
    ctb = cont_b.reshape(1, H)
    ctg = cont_ln_g.reshape(1, H)
    ctbeta = cont_ln_b.reshape(1, H)

    row_spec = pl.BlockSpec((R, D), lambda i: (i, 0))
    full = lambda shape: pl.BlockSpec(shape, lambda i: tuple(0 for _ in shape))

    X = None
    for h in range(NSPLIT):
        gA, gT, gK = halves[h]
        vec_spec = pl.BlockSpec((1, 1, R), lambda i, h=h: (h * G2 + i, 0, 0))
        out_spec = pl.BlockSpec((R, 2 * H), lambda i, h=h: (h * G2 + i, 0))
        body = _tc_body if h == 0 else _tc_body_acc
        in_specs = [
            row_spec, row_spec, row_spec,
            vec_spec, vec_spec, vec_spec, vec_spec, vec_spec,
            full((8, D)), full((D, H)), full((D, H)), full((D, H)),
            full((D, H)), full((1, H)), full((1, H)), full((1, H)),
            full((8, H)), full((1, H)), full((1, H)), full((1, H)),
        ]
        args = [gA, gT, gK, ii, xel, xua, xia, xta, embI, W0, W1, W2, W3,
                cb, cg, cbeta, cWp, ctb, ctg, ctbeta]
        kwargs = {}
        if h > 0:
            in_specs.append(pl.BlockSpec(memory_space=pl.ANY))
            args.append(X)
            kwargs = dict(input_output_aliases={len(args) - 1: 0})
        X = pl.pallas_call(
            body,
            grid=(G2,),
            in_specs=in_specs,
            out_specs=out_spec,
            out_shape=jax.ShapeDtypeStruct((N, 2 * H), jnp.float32),
            **kwargs,
        )(*args)

    return X.reshape(S, B, 2 * H).transpose(1, 0, 2)


def _tc_body_acc(gA, gT, gK, ii, xel, xua, xia, xta, embI, W0, W1, W2, W3,
                 cb, cg, cbeta, cW, ctb, ctg, ctbeta, prev, out):
    _tc_body(gA, gT, gK, ii, xel, xua, xia, xta, embI, W0, W1, W2, W3,
             cb, cg, cbeta, cW, ctb, ctg, ctbeta, out)


# speedup vs baseline: 1.0025x; 1.0025x over previous
import functools

import jax
import jax.numpy as jnp
from jax import lax
from jax.experimental import pallas as pl
from jax.experimental.pallas import tpu as pltpu
from jax.experimental.pallas import tpu_sc as plsc

B, S = 1024, 50
N = B * S
D = 128
H = 256

NC, NS = 2, 16
NW = NC * NS
NSPLIT = 2
NH = N // NSPLIT
ROWS_PER_W = NH // NW
CHUNK = 80
NCHUNK = ROWS_PER_W // CHUNK


def _sc_gather3(tabA, tabT, tabK, idxA, idxT, idxK):
    mesh = plsc.VectorSubcoreMesh(core_axis_name="c", subcore_axis_name="s")

    NBUF = 3

    @functools.partial(
        pl.kernel,
        mesh=mesh,
        out_type=[jax.ShapeDtypeStruct((NH, D), jnp.float32)] * 3,
        scratch_types=(
            [pltpu.VMEM((NCHUNK, CHUNK), jnp.int32)] * 3
            + [pltpu.VMEM((CHUNK, D), jnp.float32)] * (3 * NBUF)
            + [pltpu.SemaphoreType.DMA] * (6 * NBUF)
            + [pltpu.VMEM_SHARED((1001, D), jnp.float32)] * 2
        ),
    )
    def k(tabA_h, tabT_h, tabK_h, idxA_h, idxT_h, idxK_h,
          outA_h, outT_h, outK_h, *scr):
        idxv = scr[0:3]
        bufs = [scr[3 + t * NBUF: 3 + (t + 1) * NBUF] for t in range(3)]
        o = 3 + 3 * NBUF
        gsem = [scr[o + t * NBUF: o + (t + 1) * NBUF] for t in range(3)]
        o += 3 * NBUF
        wsem = [scr[o + t * NBUF: o + (t + 1) * NBUF] for t in range(3)]
        spT, spK = scr[o + 3 * NBUF:]

        wid = lax.axis_index("s") * NC + lax.axis_index("c")
        base = wid * ROWS_PER_W

        @pl.when(lax.axis_index("s") == 0)
        def _stage():
            pltpu.sync_copy(tabT_h, spT)
            pltpu.sync_copy(tabK_h, spK)

        pltpu.sync_copy(idxA_h.at[wid], idxv[0])
        pltpu.sync_copy(idxT_h.at[wid], idxv[1])
        pltpu.sync_copy(idxK_h.at[wid], idxv[2])
        plsc.subcore_barrier()

        tabs = (tabA_h, spT, spK)
        outs = (outA_h, outT_h, outK_h)

        def g_cp(t, j, s):
            return pltpu.make_async_copy(
                tabs[t].at[idxv[t].at[j]], bufs[t][s], gsem[t][s])

        def w_cp(t, j, s):
            return pltpu.make_async_copy(
                bufs[t][s], outs[t].at[pl.ds(base + j * CHUNK, CHUNK)],
                wsem[t][s])

        for j in range(NBUF):
            for t in range(3):
                g_cp(t, j, j).start()
        for j in range(NCHUNK):
            s = j % NBUF
            for t in range(3):
                g_cp(t, j, s).wait()
                w_cp(t, j, s).start()
            if j + NBUF < NCHUNK:
                for t in range(3):
                    w_cp(t, j, s).wait()
                    g_cp(t, j + NBUF, s).start()
        for j in range(max(0, NCHUNK - NBUF), NCHUNK):
            for t in range(3):
                w_cp(t, j, j % NBUF).wait()

    return k(tabA, tabT, tabK, idxA, idxT, idxK)


R = 2560
G2 = NH // R


_DN_T = (((0,), (0,)), ((), ()))


def _tc_body(gA, gT, gK, ii, xel, xua, xia, xta, embI, W0, W1, W2, W3,
             cb, cg, cbeta, cW, ctb, ctg, ctbeta, out):
    eps = 1e-5
    acc = jnp.dot(gA[...], W1[...], preferred_element_type=jnp.float32)
    acc += jnp.dot(gT[...], W2[...], preferred_element_type=jnp.float32)
    acc += jnp.dot(gK[...], W3[...], preferred_element_type=jnp.float32)
    P0 = jnp.dot(embI[...], W0[...], preferred_element_type=jnp.float32)
    iif = ii[0].astype(jnp.float32)
    s0 = jnp.maximum(0.0, 1.0 - jnp.abs(iif))
    s1 = jnp.maximum(0.0, 1.0 - jnp.abs(iif - 1.0))
    s2 = jnp.maximum(0.0, 1.0 - jnp.abs(iif - 2.0))
    oh = jnp.concatenate(
        [s0, s1, s2, jnp.zeros((5, s0.shape[1]), jnp.float32)], axis=0)
    p = lax.dot_general(oh, P0, _DN_T, preferred_element_type=jnp.float32)
    x = acc + p + cb[...]
    mu = jnp.mean(x, axis=-1, keepdims=True)
    xc = x - mu
    var = jnp.mean(xc * xc, axis=-1, keepdims=True)
    cate = xc * lax.rsqrt(var + eps) * cg[...] + cbeta[...]

    xq = jnp.concatenate(
        [xel[0], xua[0], xia[0], xta[0],
         jnp.zeros((4, xel.shape[2]), jnp.float32)], axis=0)
    y = lax.dot_general(xq, cW[...], _DN_T,
                        preferred_element_type=jnp.float32) + ctb[...]
    muy = jnp.mean(y, axis=-1, keepdims=True)
    yc = y - muy
    vary = jnp.mean(yc * yc, axis=-1, keepdims=True)
    cont = yc * lax.rsqrt(vary + eps) * ctg[...] + ctbeta[...]

    out[...] = jnp.concatenate([cate, cont], axis=-1)


def kernel(data_assessmentItemID, data_testId, data_KnowledgeTag, data_elapsed,
           data_user_acc, data_item_acc, data_tag_acc, data_answerCode,
           data_mask, data_interaction, emb_interaction, emb_assessmentItemID,
           emb_testId, emb_KnowledgeTag, comb_W, comb_b, comb_ln_g, comb_ln_b,
           cont_W, cont_b, cont_ln_g, cont_ln_b):
    idxA = data_assessmentItemID.T.reshape(NSPLIT, NW, NCHUNK, CHUNK).astype(jnp.int32)
    idxT = data_testId.T.reshape(NSPLIT, NW, NCHUNK, CHUNK).astype(jnp.int32)
    idxK = data_KnowledgeTag.T.reshape(NSPLIT, NW, NCHUNK, CHUNK).astype(jnp.int32)

    halves = [
        _sc_gather3(emb_assessmentItemID, emb_testId, emb_KnowledgeTag,
                    idxA[h], idxT[h], idxK[h])
        for h in range(NSPLIT)
    ]

    ii = data_interaction.T.reshape(NSPLIT * G2, 1, R).astype(jnp.int32)
    xel = data_elapsed.T.reshape(NSPLIT * G2, 1, R)
    xua = data_user_acc.T.reshape(NSPLIT * G2, 1, R)
    xia = data_item_acc.T.reshape(NSPLIT * G2, 1, R)
    xta = data_tag_acc.T.reshape(NSPLIT * G2, 1, R)

    embI = jnp.zeros((8, D), jnp.float32).at[0:3].set(emb_interaction)
    W0 = comb_W[0:D]
    W1 = comb_W[D:2 * D]
    W2 = comb_W[2 * D:3 * D]
    W3 = comb_W[3 * D:4 * D]
    cb = comb_b.reshape(1, H)
    cg = comb_ln_g.reshape(1, H)
    cbeta = comb_ln_b.reshape(1, H)
    cWp = jnp.zeros((8, H), jnp.float32).at[0:4].set(cont_W)
    ctb = cont_b.reshape(1, H)
    ctg = cont_ln_g.reshape(1, H)
    ctbeta = cont_ln_b.reshape(1, H)

    row_spec = pl.BlockSpec((R, D), lambda i: (i, 0))
    full = lambda shape: pl.BlockSpec(shape, lambda i: tuple(0 for _ in shape))

    X = None
    for h in range(NSPLIT):
        gA, gT, gK = halves[h]
        vec_spec = pl.BlockSpec((1, 1, R), lambda i, h=h: (h * G2 + i, 0, 0))
        out_spec = pl.BlockSpec((R, 2 * H), lambda i, h=h: (h * G2 + i, 0))
        body = _tc_body if h == 0 else _tc_body_acc
        in_specs = [
            row_spec, row_spec, row_spec,
            vec_spec, vec_spec, vec_spec, vec_spec, vec_spec,
            full((8, D)), full((D, H)), full((D, H)), full((D, H)),
            full((D, H)), full((1, H)), full((1, H)), full((1, H)),
            full((8, H)), full((1, H)), full((1, H)), full((1, H)),
        ]
        args = [gA, gT, gK, ii, xel, xua, xia, xta, embI, W0, W1, W2, W3,
                cb, cg, cbeta, cWp, ctb, ctg, ctbeta]
        kwargs = {}
        if h > 0:
            in_specs.append(pl.BlockSpec(memory_space=pl.ANY))
            args.append(X)
            kwargs = dict(input_output_aliases={len(args) - 1: 0})
        X = pl.pallas_call(
            body,
            grid=(G2,),
            in_specs=in_specs,
            out_specs=out_spec,
            out_shape=jax.ShapeDtypeStruct((N, 2 * H), jnp.float32),
            **kwargs,
        )(*args)

    return X.reshape(S, B, 2 * H).transpose(1, 0, 2)


def _tc_body_acc(gA, gT, gK, ii, xel, xua, xia, xta, embI, W0, W1, W2, W3,
                 cb, cg, cbeta, cW, ctb, ctg, ctbeta, prev, out):
    _tc_body(gA, gT, gK, ii, xel, xua, xia, xta, embI, W0, W1, W2, W3,
             cb, cg, cbeta, cW, ctb, ctg, ctbeta, out)
